# baseline (speedup 1.0000x reference)
import functools

import jax
import jax.numpy as jnp
from jax import lax
from jax.experimental import pallas as pl
from jax.experimental.pallas import tpu as pltpu
from jax.experimental.pallas import tpu_sc as plsc

NUM_CORES = 2
NUM_SUBCORES = 16
NW = NUM_CORES * NUM_SUBCORES
BATCH = 16384
CHUNK = 64
CH = BATCH // (NW * CHUNK)

_mesh = plsc.VectorSubcoreMesh(core_axis_name="c", subcore_axis_name="s")


@functools.partial(
    pl.kernel,
    mesh=_mesh,
    out_type=jax.ShapeDtypeStruct((NW, CH, CHUNK), jnp.float32),
    scratch_types=[
        pltpu.VMEM((CH, CHUNK), jnp.int32),
        pltpu.VMEM((CH, CHUNK), jnp.float32),
        pltpu.SemaphoreType.DMA,
        pltpu.SemaphoreType.DMA,
        pltpu.SemaphoreType.DMA,
    ],
)
def _gather_kernel(table_hbm, idx_hbm, out_hbm, idx_v, vals_v,
                   sem_i, sem_g, sem_w):
    wid = lax.axis_index("s") * NUM_CORES + lax.axis_index("c")
    idx_copies = [
        pltpu.async_copy(idx_hbm.at[wid, j], idx_v.at[j], sem_i)
        for j in range(CH)
    ]
    gathers = []
    for j in range(CH):
        idx_copies[j].wait()
        gathers.append(
            pltpu.async_copy(table_hbm.at[idx_v.at[j]], vals_v.at[j], sem_g)
        )
    writes = []
    for j in range(CH):
        gathers[j].wait()
        writes.append(
            pltpu.async_copy(vals_v.at[j], out_hbm.at[wid, j], sem_w)
        )
    for w in writes:
        w.wait()


def kernel(weights, indices):
    idx = jnp.asarray(indices, jnp.int32).reshape(NW, CH, CHUNK)
    out = _gather_kernel(weights, idx)
    return out.reshape(BATCH)


# speedup vs baseline: 1.0970x; 1.0970x over previous
import functools

import jax
import jax.numpy as jnp
from jax import lax
from jax.experimental import pallas as pl
from jax.experimental.pallas import tpu as pltpu
from jax.experimental.pallas import tpu_sc as plsc

NUM_CORES = 2
NUM_SUBCORES = 16
NW = NUM_CORES * NUM_SUBCORES
BATCH = 16384
CHUNK = 128
CH = BATCH // (NW * CHUNK)

_mesh = plsc.VectorSubcoreMesh(core_axis_name="c", subcore_axis_name="s")


@functools.partial(
    pl.kernel,
    mesh=_mesh,
    out_type=jax.ShapeDtypeStruct((NW, CH, CHUNK), jnp.float32),
    scratch_types=[
        pltpu.VMEM((CH, CHUNK), jnp.int32),
        pltpu.VMEM((CH, CHUNK), jnp.float32),
        pltpu.SemaphoreType.DMA,
    ],
)
def _gather_kernel(table_hbm, idx_hbm, out_hbm, idx_v, vals_v, sem):
    wid = lax.axis_index("s") * NUM_CORES + lax.axis_index("c")
    pltpu.sync_copy(idx_hbm.at[wid], idx_v)
    copies = [
        pltpu.async_copy(table_hbm.at[idx_v.at[j]], vals_v.at[j], sem)
        for j in range(CH)
    ]
    for c in copies:
        c.wait()
    pltpu.sync_copy(vals_v, out_hbm.at[wid])


def kernel(weights, indices):
    idx = jnp.asarray(indices, jnp.int32).reshape(NW, CH, CHUNK)
    out = _gather_kernel(weights, idx)
    return out.reshape(BATCH)
